# baseline (speedup 1.0000x reference)
import functools

import jax
import jax.numpy as jnp
from jax.experimental import pallas as pl
from jax.experimental.pallas import tpu as pltpu

_NUM_SAMPLES = 512
_RADIUS2 = 0.2 ** 2
_K = 32
_EMBED = 128
_BIG = 1e30


def _fps_body(xt_ref, fpt_ref, cen_ref, d_scr):
    B, _, N = xt_ref.shape
    M = cen_ref.shape[1]
    iota = jax.lax.broadcasted_iota(jnp.int32, (B, N), 1)

    c = fpt_ref[:, :]
    cen_ref[:, 0:1, :] = c[:, None, :]
    d = ((xt_ref[:, 0, :] - c[:, 0:1]) ** 2
         + (xt_ref[:, 1, :] - c[:, 1:2]) ** 2
         + (xt_ref[:, 2, :] - c[:, 2:3]) ** 2)
    d_scr[:, :] = d

    def step(t, _):
        d = d_scr[:, :]
        m = jnp.max(d, axis=1, keepdims=True)
        sel = jnp.where(d == m, iota, N)
        idx = jnp.min(sel, axis=1, keepdims=True)
        oh = (iota == idx).astype(jnp.float32)
        c0 = jnp.sum(oh * xt_ref[:, 0, :], axis=1, keepdims=True)
        c1 = jnp.sum(oh * xt_ref[:, 1, :], axis=1, keepdims=True)
        c2 = jnp.sum(oh * xt_ref[:, 2, :], axis=1, keepdims=True)
        cen_ref[:, pl.ds(t, 1), :] = jnp.concatenate([c0, c1, c2], axis=1)[:, None, :]
        nd = ((xt_ref[:, 0, :] - c0) ** 2
              + (xt_ref[:, 1, :] - c1) ** 2
              + (xt_ref[:, 2, :] - c2) ** 2)
        d_scr[:, :] = jnp.minimum(d, nd)
        return 0

    jax.lax.fori_loop(1, M, step, 0)


def _group_mlp_body(xt_ref, xtn_ref, cen_ref, w1_ref, b1_ref, w2_ref, b2_ref,
                    out_ref, nb_scr):
    _, _, N = xt_ref.shape
    R = cen_ref.shape[1]
    xt = xt_ref[0]
    cen = cen_ref[0]
    iota = jax.lax.broadcasted_iota(jnp.int32, (R, N), 1)

    d = ((xt[0, :][None, :] - cen[:, 0:1]) ** 2
         + (xt[1, :][None, :] - cen[:, 1:2]) ** 2
         + (xt[2, :][None, :] - cen[:, 2:3]) ** 2)
    dm = jnp.where(d < _RADIUS2, d, _BIG)
    xlast = xtn_ref[0, N - 1, :]
    xtn = xtn_ref[0]

    def extract(k, dm):
        m = jnp.min(dm, axis=1, keepdims=True)
        sel = jnp.where(dm == m, iota, N)
        idx = jnp.min(sel, axis=1, keepdims=True)
        oh = (iota == idx).astype(jnp.float32)
        coords = jax.lax.dot_general(
            oh, xtn, (((1,), (0,)), ((), ())),
            precision=jax.lax.Precision.HIGHEST,
            preferred_element_type=jnp.float32)
        valid = m < _BIG
        coords = jnp.where(valid, coords, xlast[None, :])
        nb_scr[pl.ds(k * R, R), :] = coords - cen
        return jnp.where(oh > 0, _BIG, dm)

    jax.lax.fori_loop(0, _K, extract, dm)

    nb = nb_scr[:, :]
    h = jax.lax.dot_general(nb, w1_ref[:, :], (((1,), (0,)), ((), ())),
                            precision=jax.lax.Precision.HIGHEST,
                            preferred_element_type=jnp.float32)
    h = jnp.maximum(h + b1_ref[:][None, :], 0.0)
    h = jax.lax.dot_general(h, w2_ref[:, :], (((1,), (0,)), ((), ())),
                            precision=jax.lax.Precision.HIGHEST,
                            preferred_element_type=jnp.float32)
    h = jnp.maximum(h + b2_ref[:][None, :], 0.0)
    out_ref[0] = jnp.max(h.reshape(_K, R, _EMBED), axis=0)


@jax.jit
def kernel(x, W1, b1, W2, b2):
    B, N, D = x.shape
    M = _NUM_SAMPLES
    R = 8

    xt = jnp.transpose(x, (0, 2, 1))
    first_idx = jax.random.randint(jax.random.PRNGKey(0), (B,), 0, N)
    first_pts = x[jnp.arange(B), first_idx]

    centers = pl.pallas_call(
        _fps_body,
        out_shape=jax.ShapeDtypeStruct((B, M, 3), jnp.float32),
        in_specs=[
            pl.BlockSpec((B, D, N), lambda: (0, 0, 0)),
            pl.BlockSpec((B, D), lambda: (0, 0)),
        ],
        out_specs=pl.BlockSpec((B, M, 3), lambda: (0, 0, 0)),
        scratch_shapes=[pltpu.VMEM((B, N), jnp.float32)],
    )(xt, first_pts)

    out = pl.pallas_call(
        _group_mlp_body,
        grid=(B, M // R),
        out_shape=jax.ShapeDtypeStruct((B, M, _EMBED), jnp.float32),
        in_specs=[
            pl.BlockSpec((1, D, N), lambda b, i: (b, 0, 0)),
            pl.BlockSpec((1, N, D), lambda b, i: (b, 0, 0)),
            pl.BlockSpec((1, R, D), lambda b, i: (b, i, 0)),
            pl.BlockSpec((D, _EMBED), lambda b, i: (0, 0)),
            pl.BlockSpec((_EMBED,), lambda b, i: (0,)),
            pl.BlockSpec((_EMBED, _EMBED), lambda b, i: (0, 0)),
            pl.BlockSpec((_EMBED,), lambda b, i: (0,)),
        ],
        out_specs=pl.BlockSpec((1, R, _EMBED), lambda b, i: (b, i, 0)),
        scratch_shapes=[pltpu.VMEM((_K * R, 3), jnp.float32)],
    )(xt, x, centers, W1, b1, W2, b2)

    return out


# speedup vs baseline: 20.0697x; 20.0697x over previous
import functools

import jax
import jax.numpy as jnp
from jax import lax
from jax.experimental import pallas as pl
from jax.experimental.pallas import tpu as pltpu
from jax.experimental.pallas import tpu_sc as plsc

_B = 8
_N = 4096
_NUM_SAMPLES = 512
_RADIUS2 = 0.2 ** 2
_K = 32
_EMBED = 128
_BIG = 1e30

_NC, _NS = 2, 16
_NW = _NC * _NS
_ROWS_PER_W = _B * _NUM_SAMPLES // _NW
_WPB = _NUM_SAMPLES // _ROWS_PER_W
_CAND = _N + 64


def _fps_body(xt_ref, fpt_ref, cen_ref, d_scr):
    B, _, N = xt_ref.shape
    M = cen_ref.shape[1]
    iota = jax.lax.broadcasted_iota(jnp.int32, (B, N), 1)

    c = fpt_ref[:, :]
    cen_ref[:, 0:1, :] = c[:, None, :]
    d = ((xt_ref[:, 0, :] - c[:, 0:1]) ** 2
         + (xt_ref[:, 1, :] - c[:, 1:2]) ** 2
         + (xt_ref[:, 2, :] - c[:, 2:3]) ** 2)
    d_scr[:, :] = d

    def step(t, _):
        d = d_scr[:, :]
        m = jnp.max(d, axis=1, keepdims=True)
        sel = jnp.where(d == m, iota, N)
        idx = jnp.min(sel, axis=1, keepdims=True)
        oh = (iota == idx).astype(jnp.float32)
        c0 = jnp.sum(oh * xt_ref[:, 0, :], axis=1, keepdims=True)
        c1 = jnp.sum(oh * xt_ref[:, 1, :], axis=1, keepdims=True)
        c2 = jnp.sum(oh * xt_ref[:, 2, :], axis=1, keepdims=True)
        cen_ref[:, pl.ds(t, 1), :] = jnp.concatenate([c0, c1, c2], axis=1)[:, None, :]
        nd = ((xt_ref[:, 0, :] - c0) ** 2
              + (xt_ref[:, 1, :] - c1) ** 2
              + (xt_ref[:, 2, :] - c2) ** 2)
        d_scr[:, :] = jnp.minimum(d, nd)
        return 0

    jax.lax.fori_loop(1, M, step, 0)


def _sc_select_body(xx_ref, xy_ref, xz_ref, cen_ref, out_ref,
                    xq_x, xq_y, xq_z, cen_v, cand_d, cand_i, sel_i, outb):
    wid = lax.axis_index("s") * _NC + lax.axis_index("c")
    b = wid // _WPB
    base_m = (wid % _WPB) * _ROWS_PER_W

    pltpu.sync_copy(xx_ref.at[pl.ds(b * _N, _N)], xq_x)
    pltpu.sync_copy(xy_ref.at[pl.ds(b * _N, _N)], xq_y)
    pltpu.sync_copy(xz_ref.at[pl.ds(b * _N, _N)], xq_z)
    pltpu.sync_copy(cen_ref.at[pl.ds((b * _NUM_SAMPLES + base_m) * 3,
                                     _ROWS_PER_W * 3)], cen_v)

    iota16 = lax.iota(jnp.int32, 16)
    big = jnp.full((16,), _BIG, jnp.float32)
    nlast = jnp.full((16,), _N - 1, jnp.int32)

    def row_body(r, _):
        r3 = jnp.full((16,), r * 3, jnp.int32)
        cx = plsc.load_gather(cen_v, [r3])
        cy = plsc.load_gather(cen_v, [r3 + 1])
        cz = plsc.load_gather(cen_v, [r3 + 2])

        def chunk(j, cnt):
            base = j * 16
            dx = xq_x[pl.ds(base, 16)] - cx
            dy = xq_y[pl.ds(base, 16)] - cy
            dz = xq_z[pl.ds(base, 16)] - cz
            dsq = dx * dx + dy * dy + dz * dz
            mask = dsq < _RADIUS2
            plsc.store_compressed(cand_d.at[pl.ds(cnt, 16)], dsq, mask=mask)
            plsc.store_compressed(cand_i.at[pl.ds(cnt, 16)], iota16 + base,
                                  mask=mask)
            return cnt + jnp.sum(mask.astype(jnp.int32))

        cnt = lax.fori_loop(0, _N // 16, chunk, jnp.int32(0))
        cand_d[pl.ds(cnt, 16)] = big
        nv = (cnt + 15) // 16

        def tour(j, carry):
            b0, b1 = carry
            vs = lax.sort(cand_d[pl.ds(j * 16, 16)])
            m1 = jnp.minimum(b1, lax.rev(vs, (0,)))
            lo = jnp.minimum(b0, m1)
            hi = jnp.maximum(b0, m1)
            return lax.sort(lo), lax.sort(hi)

        _, b1 = lax.fori_loop(0, nv, tour, (big, big))
        tau = jnp.max(b1)

        sel_i[pl.ds(0, 16)] = nlast
        sel_i[pl.ds(16, 16)] = nlast

        def pick(j, off):
            v = cand_d[pl.ds(j * 16, 16)]
            iv = cand_i[pl.ds(j * 16, 16)]
            mask = (v <= tau) & (v < _RADIUS2)
            offc = jnp.minimum(off, 32)
            plsc.store_compressed(sel_i.at[pl.ds(offc, 16)], iv, mask=mask)
            return off + jnp.sum(mask.astype(jnp.int32))

        lax.fori_loop(0, nv, pick, jnp.int32(0))

        orow = r * _K
        for t in range(2):
            idxv = sel_i[pl.ds(t * 16, 16)]
            gx = plsc.load_gather(xq_x, [idxv]) - cx
            gy = plsc.load_gather(xq_y, [idxv]) - cy
            gz = plsc.load_gather(xq_z, [idxv]) - cz
            flat = (orow + t * 16 + iota16) * 3
            plsc.store_scatter(outb, [flat], gx)
            plsc.store_scatter(outb, [flat + 1], gy)
            plsc.store_scatter(outb, [flat + 2], gz)
        return 0

    lax.fori_loop(0, _ROWS_PER_W, row_body, 0)
    nout = _ROWS_PER_W * _K * 3
    pltpu.sync_copy(outb, out_ref.at[pl.ds(wid * nout, nout)])


def _mlp_body(nb_ref, w1_ref, b1_ref, w2_ref, b2_ref, out_ref):
    RC = out_ref.shape[0]
    h = jax.lax.dot_general(nb_ref[:, :], w1_ref[:, :], (((1,), (0,)), ((), ())),
                            precision=jax.lax.Precision.HIGHEST,
                            preferred_element_type=jnp.float32)
    h = jnp.maximum(h + b1_ref[:][None, :], 0.0)
    h = jax.lax.dot_general(h, w2_ref[:, :], (((1,), (0,)), ((), ())),
                            precision=jax.lax.Precision.HIGHEST,
                            preferred_element_type=jnp.float32)
    h = jnp.maximum(h + b2_ref[:][None, :], 0.0)
    out_ref[:, :] = jnp.max(h.reshape(RC, _K, _EMBED), axis=1)


@jax.jit
def kernel(x, W1, b1, W2, b2):
    B, N, D = x.shape
    M = _NUM_SAMPLES

    xt = jnp.transpose(x, (0, 2, 1))
    first_idx = jax.random.randint(jax.random.PRNGKey(0), (B,), 0, N)
    first_pts = x[jnp.arange(B), first_idx]

    centers = pl.pallas_call(
        _fps_body,
        out_shape=jax.ShapeDtypeStruct((B, M, 3), jnp.float32),
        in_specs=[
            pl.BlockSpec((B, D, N), lambda: (0, 0, 0)),
            pl.BlockSpec((B, D), lambda: (0, 0)),
        ],
        out_specs=pl.BlockSpec((B, M, 3), lambda: (0, 0, 0)),
        scratch_shapes=[pltpu.VMEM((B, N), jnp.float32)],
    )(xt, first_pts)

    mesh = plsc.VectorSubcoreMesh(core_axis_name="c", subcore_axis_name="s",
                                  num_cores=_NC, num_subcores=_NS)
    rel_flat = pl.kernel(
        _sc_select_body,
        out_type=jax.ShapeDtypeStruct((B * M * _K * 3,), jnp.float32),
        mesh=mesh,
        scratch_types=[
            pltpu.VMEM((_N,), jnp.float32),
            pltpu.VMEM((_N,), jnp.float32),
            pltpu.VMEM((_N,), jnp.float32),
            pltpu.VMEM((_ROWS_PER_W * 3,), jnp.float32),
            pltpu.VMEM((_CAND,), jnp.float32),
            pltpu.VMEM((_CAND,), jnp.int32),
            pltpu.VMEM((64,), jnp.int32),
            pltpu.VMEM((_ROWS_PER_W * _K * 3,), jnp.float32),
        ],
        compiler_params=pltpu.CompilerParams(needs_layout_passes=False),
    )(xt[:, 0, :].reshape(-1), xt[:, 1, :].reshape(-1), xt[:, 2, :].reshape(-1),
      centers.reshape(-1))
    rel = rel_flat.reshape(B * M * _K, 3)

    RC = 64
    out = pl.pallas_call(
        _mlp_body,
        grid=(B * M // RC,),
        out_shape=jax.ShapeDtypeStruct((B * M, _EMBED), jnp.float32),
        in_specs=[
            pl.BlockSpec((RC * _K, 3), lambda i: (i, 0)),
            pl.BlockSpec((D, _EMBED), lambda i: (0, 0)),
            pl.BlockSpec((_EMBED,), lambda i: (0,)),
            pl.BlockSpec((_EMBED, _EMBED), lambda i: (0, 0)),
            pl.BlockSpec((_EMBED,), lambda i: (0,)),
        ],
        out_specs=pl.BlockSpec((RC, _EMBED), lambda i: (i, 0)),
    )(rel, W1, b1, W2, b2)

    return out.reshape(B, M, _EMBED)
